# baseline (speedup 1.0000x reference)
import functools

import jax
import jax.numpy as jnp
from jax import lax
from jax.experimental import pallas as pl
from jax.experimental.pallas import tpu as pltpu
from jax.experimental.pallas import tpu_sc as plsc

VOCAB = 1000000
EMBED = 32
NUM_CLASSES = 100
BATCH = 16384
SEQ = 200

NC = 2
NS = 16
NW = NC * NS
ROWS_W = BATCH // NW
GCH = 40
CPR = SEQ // GCH
CROWS_W = ROWS_W * CPR
PCHUNK = 32
NRING = 4

_HALF = EMBED // 2


GA = 104
GB = SEQ - GA


PH_ROWS = ROWS_W // 2


def _pool_body(xi_hbm, tab_hbm, out_hbm, idx_v, rowsa_v, rowsb_v, pool_v, gsem, osem):
    wid = lax.axis_index("s") * NC + lax.axis_index("c")
    rbase = pl.multiple_of(wid * ROWS_W, ROWS_W)

    def issue(rl, s):
        pltpu.async_copy(
            tab_hbm.at[idx_v.at[rl, pl.ds(0, GA)]], rowsa_v.at[s], gsem.at[2 * s]
        )
        pltpu.async_copy(
            tab_hbm.at[idx_v.at[rl, pl.ds(GA, GB)]], rowsb_v.at[s], gsem.at[2 * s + 1]
        )

    inv = 1.0 / SEQ

    for phase in range(2):
        pltpu.sync_copy(
            xi_hbm.at[pl.ds(rbase + phase * PH_ROWS, PH_ROWS)], idx_v
        )
        for s in range(NRING):
            issue(s, s)

        @pl.loop(0, PH_ROWS, step=NRING)
        def _row(r0):
            for s in range(NRING):
                rl = r0 + s
                r = phase * PH_ROWS + rl
                pltpu.make_async_copy(
                    tab_hbm.at[idx_v.at[rl, pl.ds(0, GA)]],
                    rowsa_v.at[s],
                    gsem.at[2 * s],
                ).wait()
                pltpu.make_async_copy(
                    tab_hbm.at[idx_v.at[rl, pl.ds(GA, GB)]],
                    rowsb_v.at[s],
                    gsem.at[2 * s + 1],
                ).wait()

                a0 = rowsa_v[s, 0, 0:_HALF]
                a1 = rowsa_v[s, 0, _HALF:EMBED]
                c0 = rowsa_v[s, 1, 0:_HALF]
                c1 = rowsa_v[s, 1, _HALF:EMBED]
                for l in range(2, GA, 2):
                    a0 = a0 + rowsa_v[s, l, 0:_HALF]
                    a1 = a1 + rowsa_v[s, l, _HALF:EMBED]
                    c0 = c0 + rowsa_v[s, l + 1, 0:_HALF]
                    c1 = c1 + rowsa_v[s, l + 1, _HALF:EMBED]
                for l in range(0, GB, 2):
                    a0 = a0 + rowsb_v[s, l, 0:_HALF]
                    a1 = a1 + rowsb_v[s, l, _HALF:EMBED]
                    c0 = c0 + rowsb_v[s, l + 1, 0:_HALF]
                    c1 = c1 + rowsb_v[s, l + 1, _HALF:EMBED]

                @pl.when(rl + NRING < PH_ROWS)
                def _():
                    issue(rl + NRING, s)

                s0 = (a0 + c0) * inv
                s1 = (a1 + c1) * inv
                pbuf = (r // PCHUNK) % 2
                slot = r % PCHUNK
                pool_v[pbuf, slot, 0:_HALF] = s0
                pool_v[pbuf, slot, _HALF:EMBED] = s1

                @pl.when(slot == PCHUNK - 1)
                def _flush():
                    @pl.when(r >= 2 * PCHUNK - 1)
                    def _():
                        pltpu.make_async_copy(
                            pool_v.at[0], out_hbm.at[pl.ds(rbase, PCHUNK)], osem
                        ).wait()

                    pltpu.async_copy(
                        pool_v.at[pbuf],
                        out_hbm.at[
                            pl.ds(
                                pl.multiple_of(rbase + r + 1 - PCHUNK, PCHUNK),
                                PCHUNK,
                            )
                        ],
                        osem,
                    )

    pltpu.make_async_copy(
        pool_v.at[0], out_hbm.at[pl.ds(rbase, PCHUNK)], osem
    ).wait()


_pool = functools.partial(
    pl.kernel,
    out_type=jax.ShapeDtypeStruct((BATCH, EMBED), jnp.float32),
    mesh=plsc.VectorSubcoreMesh(
        core_axis_name="c", subcore_axis_name="s", num_cores=NC, num_subcores=NS
    ),
    scratch_types=[
        pltpu.VMEM((ROWS_W // 2, SEQ), jnp.int32),
        pltpu.VMEM((NRING, GA, EMBED), jnp.float32),
        pltpu.VMEM((NRING, GB, EMBED), jnp.float32),
        pltpu.VMEM((2, PCHUNK, EMBED), jnp.float32),
        pltpu.SemaphoreType.DMA((2 * NRING,)),
        pltpu.SemaphoreType.DMA,
    ],
    compiler_params=pltpu.CompilerParams(use_tc_tiling_on_sc=False),
)(_pool_body)


_TW = 8192
_TGRID = (VOCAB + _TW - 1) // _TW


def _tab_tr_body(in_ref, out_ref):
    t = in_ref[...].T.reshape(_TW // 4, 4, EMBED)
    out_ref[...] = jnp.concatenate([t[:, a, :] for a in range(4)], axis=1)


_tab_tr = pl.pallas_call(
    _tab_tr_body,
    grid=(_TGRID,),
    in_specs=[pl.BlockSpec((EMBED, _TW), lambda i: (0, i))],
    out_specs=pl.BlockSpec((_TW // 4, 4 * EMBED), lambda i: (i, 0)),
    out_shape=jax.ShapeDtypeStruct((VOCAB // 4, 4 * EMBED), jnp.float32),
)


def _mlp_body(p_ref, w1_ref, b1_ref, w2_ref, b2_ref, o_ref):
    h = jnp.dot(p_ref[...], w1_ref[...], preferred_element_type=jnp.float32)
    h = jnp.maximum(h + b1_ref[...], 0.0)
    o_ref[...] = (
        jnp.dot(h, w2_ref[...], preferred_element_type=jnp.float32) + b2_ref[...]
    )


_MB = 2048

_mlp = pl.pallas_call(
    _mlp_body,
    grid=(BATCH // _MB,),
    in_specs=[
        pl.BlockSpec((_MB, EMBED), lambda i: (i, 0)),
        pl.BlockSpec((EMBED, 2 * EMBED), lambda i: (0, 0)),
        pl.BlockSpec((1, 2 * EMBED), lambda i: (0, 0)),
        pl.BlockSpec((2 * EMBED, NUM_CLASSES), lambda i: (0, 0)),
        pl.BlockSpec((1, NUM_CLASSES), lambda i: (0, 0)),
    ],
    out_specs=pl.BlockSpec((_MB, NUM_CLASSES), lambda i: (i, 0)),
    out_shape=jax.ShapeDtypeStruct((BATCH, NUM_CLASSES), jnp.float32),
)


def kernel(x, emb_table, W1, b1, W2, b2):
    tab = _tab_tr(emb_table.T).reshape(VOCAB, EMBED)
    pooled = _pool(x, tab)
    return _mlp(
        pooled,
        W1.T,
        b1.reshape(1, 2 * EMBED),
        W2.T,
        b2.reshape(1, NUM_CLASSES),
    )


# speedup vs baseline: 1.2065x; 1.2065x over previous
import functools

import jax
import jax.numpy as jnp
from jax import lax
from jax.experimental import pallas as pl
from jax.experimental.pallas import tpu as pltpu
from jax.experimental.pallas import tpu_sc as plsc

VOCAB = 1000000
EMBED = 32
NUM_CLASSES = 100
BATCH = 16384
SEQ = 200

NC = 2
NS = 16
NW = NC * NS
ROWS_W = BATCH // NW
GCH = 40
CPR = SEQ // GCH
CROWS_W = ROWS_W * CPR
PCHUNK = 64
NRING = 2

_HALF = EMBED // 2


GA = 104
GB = SEQ - GA


PH_ROWS = ROWS_W


def _pool_body(xi_hbm, tab_hbm, out_hbm, idx_v, rowsa_v, rowsb_v, pool_v, gsem, osem):
    wid = lax.axis_index("s") * NC + lax.axis_index("c")
    rbase = pl.multiple_of(wid * ROWS_W, ROWS_W)

    def issue(rl, s):
        pltpu.async_copy(
            tab_hbm.at[idx_v.at[rl, pl.ds(0, GA)]], rowsa_v.at[s], gsem.at[2 * s]
        )
        pltpu.async_copy(
            tab_hbm.at[idx_v.at[rl, pl.ds(GA, GB)]], rowsb_v.at[s], gsem.at[2 * s + 1]
        )

    inv = 1.0 / SEQ

    for phase in range(1):
        pltpu.sync_copy(
            xi_hbm.at[pl.ds(rbase + phase * PH_ROWS, PH_ROWS)], idx_v
        )
        for s in range(NRING):
            issue(s, s)

        @pl.loop(0, PH_ROWS, step=NRING)
        def _row(r0):
            for s in range(NRING):
                rl = r0 + s
                r = phase * PH_ROWS + rl
                pltpu.make_async_copy(
                    tab_hbm.at[idx_v.at[rl, pl.ds(0, GA)]],
                    rowsa_v.at[s],
                    gsem.at[2 * s],
                ).wait()
                pltpu.make_async_copy(
                    tab_hbm.at[idx_v.at[rl, pl.ds(GA, GB)]],
                    rowsb_v.at[s],
                    gsem.at[2 * s + 1],
                ).wait()

                a0 = rowsa_v[s, 0, 0:_HALF]
                a1 = rowsa_v[s, 0, _HALF:EMBED]
                c0 = rowsa_v[s, 1, 0:_HALF]
                c1 = rowsa_v[s, 1, _HALF:EMBED]
                for l in range(2, GA, 2):
                    a0 = a0 + rowsa_v[s, l, 0:_HALF]
                    a1 = a1 + rowsa_v[s, l, _HALF:EMBED]
                    c0 = c0 + rowsa_v[s, l + 1, 0:_HALF]
                    c1 = c1 + rowsa_v[s, l + 1, _HALF:EMBED]
                for l in range(0, GB, 2):
                    a0 = a0 + rowsb_v[s, l, 0:_HALF]
                    a1 = a1 + rowsb_v[s, l, _HALF:EMBED]
                    c0 = c0 + rowsb_v[s, l + 1, 0:_HALF]
                    c1 = c1 + rowsb_v[s, l + 1, _HALF:EMBED]

                @pl.when(rl + NRING < PH_ROWS)
                def _():
                    issue(rl + NRING, s)

                s0 = (a0 + c0) * inv
                s1 = (a1 + c1) * inv
                pbuf = (r // PCHUNK) % 2
                slot = r % PCHUNK
                pool_v[pbuf, slot, 0:_HALF] = s0
                pool_v[pbuf, slot, _HALF:EMBED] = s1

                @pl.when(slot == PCHUNK - 1)
                def _flush():
                    @pl.when(r >= 2 * PCHUNK - 1)
                    def _():
                        pltpu.make_async_copy(
                            pool_v.at[0], out_hbm.at[pl.ds(rbase, PCHUNK)], osem
                        ).wait()

                    pltpu.async_copy(
                        pool_v.at[pbuf],
                        out_hbm.at[
                            pl.ds(
                                pl.multiple_of(rbase + r + 1 - PCHUNK, PCHUNK),
                                PCHUNK,
                            )
                        ],
                        osem,
                    )

    pltpu.make_async_copy(
        pool_v.at[0], out_hbm.at[pl.ds(rbase, PCHUNK)], osem
    ).wait()


_pool = functools.partial(
    pl.kernel,
    out_type=jax.ShapeDtypeStruct((BATCH, EMBED), jnp.float32),
    mesh=plsc.VectorSubcoreMesh(
        core_axis_name="c", subcore_axis_name="s", num_cores=NC, num_subcores=NS
    ),
    scratch_types=[
        pltpu.VMEM((ROWS_W, SEQ), jnp.int32),
        pltpu.VMEM((NRING, GA, EMBED), jnp.float32),
        pltpu.VMEM((NRING, GB, EMBED), jnp.float32),
        pltpu.VMEM((2, PCHUNK, EMBED), jnp.float32),
        pltpu.SemaphoreType.DMA((2 * NRING,)),
        pltpu.SemaphoreType.DMA,
    ],
    compiler_params=pltpu.CompilerParams(use_tc_tiling_on_sc=False),
)(_pool_body)


_TW = 8192
_TGRID = (VOCAB + _TW - 1) // _TW


def _tab_tr_body(in_ref, out_ref):
    t = in_ref[...].T.reshape(_TW // 4, 4, EMBED)
    out_ref[...] = jnp.concatenate([t[:, a, :] for a in range(4)], axis=1)


_tab_tr = pl.pallas_call(
    _tab_tr_body,
    grid=(_TGRID,),
    in_specs=[pl.BlockSpec((EMBED, _TW), lambda i: (0, i))],
    out_specs=pl.BlockSpec((_TW // 4, 4 * EMBED), lambda i: (i, 0)),
    out_shape=jax.ShapeDtypeStruct((VOCAB // 4, 4 * EMBED), jnp.float32),
)


def _mlp_body(p_ref, w1_ref, b1_ref, w2_ref, b2_ref, o_ref):
    h = jnp.dot(p_ref[...], w1_ref[...], preferred_element_type=jnp.float32)
    h = jnp.maximum(h + b1_ref[...], 0.0)
    o_ref[...] = (
        jnp.dot(h, w2_ref[...], preferred_element_type=jnp.float32) + b2_ref[...]
    )


_MB = 2048

_mlp = pl.pallas_call(
    _mlp_body,
    grid=(BATCH // _MB,),
    in_specs=[
        pl.BlockSpec((_MB, EMBED), lambda i: (i, 0)),
        pl.BlockSpec((EMBED, 2 * EMBED), lambda i: (0, 0)),
        pl.BlockSpec((1, 2 * EMBED), lambda i: (0, 0)),
        pl.BlockSpec((2 * EMBED, NUM_CLASSES), lambda i: (0, 0)),
        pl.BlockSpec((1, NUM_CLASSES), lambda i: (0, 0)),
    ],
    out_specs=pl.BlockSpec((_MB, NUM_CLASSES), lambda i: (i, 0)),
    out_shape=jax.ShapeDtypeStruct((BATCH, NUM_CLASSES), jnp.float32),
)


def kernel(x, emb_table, W1, b1, W2, b2):
    tab = _tab_tr(emb_table.T).reshape(VOCAB, EMBED)
    pooled = _pool(x, tab)
    return _mlp(
        pooled,
        W1.T,
        b1.reshape(1, 2 * EMBED),
        W2.T,
        b2.reshape(1, NUM_CLASSES),
    )
